# baseline (speedup 1.0000x reference)
import functools

import jax
import jax.numpy as jnp
from jax import lax
from jax.experimental import pallas as pl
from jax.experimental.pallas import tpu as pltpu
from jax.experimental.pallas import tpu_sc as plsc

N_NODES = 10000
N_EDGES = 320000

NC = 2
NS = 16
EW = N_EDGES // NS
B = 125
NB = EW // B
NBUF = 5
AHEAD = 2
NPAD = 10240
RPT = NPAD // NS

ROW_BLOCK = 2000


def _mm_body(x_ref, w_ref, o_ref, t_ref):
    x = x_ref[...]
    for c in range(NC):
        o = lax.dot_general(
            x, w_ref[c], (((1,), (1,)), ((), ())),
            preferred_element_type=jnp.float32)
        o_ref[c] = o
        t_ref[c] = o.astype(jnp.bfloat16)


def _fused_body(g0_ref, g1_ref, p0_ref, p1_ref, b_ref, w_ref, o_ref, t_ref):
    h = jnp.concatenate(
        [g0_ref[0] + p0_ref[0], g1_ref[0] + p1_ref[0]], axis=1)
    h = jnp.maximum(h + b_ref[...], 0.0)
    for c in range(NC):
        o = lax.dot_general(
            h, w_ref[c], (((1,), (1,)), ((), ())),
            preferred_element_type=jnp.float32)
        o_ref[c] = o
        t_ref[c] = o.astype(jnp.bfloat16)


def _final_body(g0_ref, g1_ref, p0_ref, p1_ref, b_ref, o_ref):
    o = jnp.concatenate(
        [g0_ref[0] + p0_ref[0], g1_ref[0] + p1_ref[0]], axis=1)
    o_ref[...] = (o + b_ref[...])[:, :40]


def _matmul_halves(x, wh):
    n, din = x.shape
    dh = wh.shape[1]
    grid = (n // ROW_BLOCK,)
    return pl.pallas_call(
        _mm_body,
        grid=grid,
        in_specs=[
            pl.BlockSpec((ROW_BLOCK, din), lambda i: (i, 0)),
            pl.BlockSpec((NC, dh, din), lambda i: (0, 0, 0)),
        ],
        out_specs=[
            pl.BlockSpec((NC, ROW_BLOCK, dh), lambda i: (0, i, 0)),
            pl.BlockSpec((NC, ROW_BLOCK, dh), lambda i: (0, i, 0)),
        ],
        out_shape=[
            jax.ShapeDtypeStruct((NC, n, dh), jnp.float32),
            jax.ShapeDtypeStruct((NC, n, dh), jnp.bfloat16),
        ],
    )(x, wh)


def _fused_update_matmul(g, p, b, wh):
    _, n, dgh = g.shape
    din = 2 * dgh
    dh = wh.shape[1]
    grid = (n // ROW_BLOCK,)
    return pl.pallas_call(
        _fused_body,
        grid=grid,
        in_specs=[
            pl.BlockSpec((1, ROW_BLOCK, dgh), lambda i: (0, i, 0)),
            pl.BlockSpec((1, ROW_BLOCK, dgh), lambda i: (1, i, 0)),
            pl.BlockSpec((1, ROW_BLOCK, dgh), lambda i: (0, i, 0)),
            pl.BlockSpec((1, ROW_BLOCK, dgh), lambda i: (1, i, 0)),
            pl.BlockSpec((1, din), lambda i: (0, 0)),
            pl.BlockSpec((NC, dh, din), lambda i: (0, 0, 0)),
        ],
        out_specs=[
            pl.BlockSpec((NC, ROW_BLOCK, dh), lambda i: (0, i, 0)),
            pl.BlockSpec((NC, ROW_BLOCK, dh), lambda i: (0, i, 0)),
        ],
        out_shape=[
            jax.ShapeDtypeStruct((NC, n, dh), jnp.float32),
            jax.ShapeDtypeStruct((NC, n, dh), jnp.bfloat16),
        ],
    )(g, g, p, p, b.reshape(1, din), wh)


def _final_update(g, p, b):
    _, n, dgh = g.shape
    din = 2 * dgh
    grid = (n // ROW_BLOCK,)
    return pl.pallas_call(
        _final_body,
        grid=grid,
        in_specs=[
            pl.BlockSpec((1, ROW_BLOCK, dgh), lambda i: (0, i, 0)),
            pl.BlockSpec((1, ROW_BLOCK, dgh), lambda i: (1, i, 0)),
            pl.BlockSpec((1, ROW_BLOCK, dgh), lambda i: (0, i, 0)),
            pl.BlockSpec((1, ROW_BLOCK, dgh), lambda i: (1, i, 0)),
            pl.BlockSpec((1, din), lambda i: (0, 0)),
        ],
        out_specs=pl.BlockSpec((ROW_BLOCK, 40), lambda i: (i, 0)),
        out_shape=jax.ShapeDtypeStruct((n, 40), jnp.float32),
    )(g, g, p, p, b.reshape(1, din))


GBUF = 4
SBUF = 2
GAHEAD = 3
_HI16 = -65536


@functools.lru_cache(maxsize=None)
def _make_sc_agg(dh):
    mesh = plsc.VectorSubcoreMesh(core_axis_name="c", subcore_axis_name="s")
    nw = dh // 32

    @functools.partial(
        pl.kernel,
        out_type=jax.ShapeDtypeStruct((NC, NPAD, dh), jnp.float32),
        mesh=mesh,
        compiler_params=pltpu.CompilerParams(
            use_tc_tiling_on_sc=False, needs_layout_passes=False),
        scratch_types=[
            pltpu.VMEM((NB, B), jnp.int32),
            pltpu.VMEM((NB, B), jnp.int32),
            pltpu.VMEM((GBUF, B, dh), jnp.bfloat16),
            pltpu.VMEM((SBUF, B, dh), jnp.float32),
            pltpu.VMEM_SHARED((NPAD, dh), jnp.float32),
        ]
        + [pltpu.SemaphoreType.DMA] * GBUF
        + [pltpu.SemaphoreType.DMA] * SBUF,
    )
    def agg(table, src_h, dst_h, zrows, out, src_v, dst_v, bf_v, f32_v,
            acc, *sems):
        sem_g = sems[:GBUF]
        sem_s = sems[GBUF:]
        cid = lax.axis_index("c")
        sid = lax.axis_index("s")
        half = table.at[cid]

        pltpu.sync_copy(zrows, acc.at[pl.ds(sid * RPT, RPT)])
        pltpu.sync_copy(src_h.at[sid], src_v)
        pltpu.sync_copy(dst_h.at[sid], dst_v)
        plsc.subcore_barrier()

        def start_gather(j, b):
            pltpu.async_copy(half.at[src_v.at[j]], bf_v.at[b], sem_g[b])

        def wait_gather(j, b):
            pltpu.make_async_copy(
                half.at[src_v.at[j]], bf_v.at[b], sem_g[b]).wait()

        def start_scatter(j, b):
            pltpu.async_copy(f32_v.at[b], acc.at[dst_v.at[j]], sem_s[b],
                             add=True)

        def wait_scatter(j, b):
            pltpu.make_async_copy(f32_v.at[b], acc.at[dst_v.at[j]],
                                  sem_s[b]).wait()

        lanes = lax.iota(jnp.int32, 16)
        cols_e = [w * 32 + 2 * lanes for w in range(nw)]
        cols_o = [w * 32 + 2 * lanes + 1 for w in range(nw)]
        himask = jnp.full((16,), _HI16, jnp.int32)

        def widen(bg, bs):
            dst2 = f32_v.at[bs]

            def crow(r, _):
                rowidx = jnp.full((16,), r, jnp.int32)
                for w in range(nw):
                    v = plsc.bitcast(
                        bf_v[bg, r, pl.ds(w * 32, 32)], jnp.int32)
                    even = plsc.bitcast(v << 16, jnp.float32)
                    odd = plsc.bitcast(v & himask, jnp.float32)
                    plsc.store_scatter(dst2, [rowidx, cols_e[w]], even)
                    plsc.store_scatter(dst2, [rowidx, cols_o[w]], odd)
                return 0

            lax.fori_loop(0, B, crow, 0)

        for b in range(GAHEAD):
            start_gather(b, b)

        unroll = GBUF

        def outer(j0, _):
            for u in range(unroll):
                j = j0 * unroll + u
                bg = u % GBUF
                bs = u % SBUF
                jn = j + GAHEAD

                @pl.when(jn < NB)
                def _():
                    start_gather(jn, (u + GAHEAD) % GBUF)

                wait_gather(j, bg)

                @pl.when(j >= SBUF)
                def _():
                    wait_scatter(j - SBUF, bs)

                widen(bg, bs)
                start_scatter(j, bs)

            return 0

        lax.fori_loop(0, NB // unroll, outer, 0)

        for j in range(NB - SBUF, NB):
            wait_scatter(j, j % SBUF)

        plsc.subcore_barrier()
        pltpu.sync_copy(acc.at[pl.ds(sid * RPT, RPT)],
                        out.at[cid, pl.ds(sid * RPT, RPT)])

    return agg


def _sc_aggregate(halves, src2, dst2, zrows):
    p = _make_sc_agg(halves.shape[2])(halves, src2, dst2, zrows)
    return p


def kernel(features, adj, W1, b1, W3, b3, W4, b4, W2, b2):
    src2 = adj[0].reshape(NS, NB, B)
    dst2 = adj[1].reshape(NS, NB, B)
    z64 = jnp.zeros((RPT, 64), jnp.float32)
    z32 = jnp.zeros((RPT, 32), jnp.float32)

    W1h = W1.reshape(NC, 64, 128)
    W3h = W3.reshape(NC, 64, 128)
    W4h = W4.reshape(NC, 64, 128)
    W2p = jnp.zeros((64, 128), jnp.float32).at[:40].set(W2).reshape(NC, 32, 128)
    b2p = jnp.zeros((64,), jnp.float32).at[:40].set(b2)

    g1, t1 = _matmul_halves(features, W1h)
    p1 = _sc_aggregate(t1, src2, dst2, z64)
    g2, t2 = _fused_update_matmul(g1, p1, b1, W3h)
    p2 = _sc_aggregate(t2, src2, dst2, z64)
    g3, t3 = _fused_update_matmul(g2, p2, b3, W4h)
    p3 = _sc_aggregate(t3, src2, dst2, z64)
    g4, t4 = _fused_update_matmul(g3, p3, b4, W2p)
    p4 = _sc_aggregate(t4, src2, dst2, z32)
    return _final_update(g4, p4, b2p)


# speedup vs baseline: 1.8454x; 1.8454x over previous
import functools

import jax
import jax.numpy as jnp
from jax import lax
from jax.experimental import pallas as pl
from jax.experimental.pallas import tpu as pltpu
from jax.experimental.pallas import tpu_sc as plsc

N_NODES = 10000
N_EDGES = 320000

NC = 2
NS = 16
EW = N_EDGES // NS
B = 125
NB = EW // B
NBUF = 5
AHEAD = 3
NPAD = 10240
RPT = NPAD // NS

ROW_BLOCK = 2000


def _mm_body(x_ref, w_ref, o_ref):
    x = x_ref[...]
    for c in range(NC):
        o_ref[c] = lax.dot_general(
            x, w_ref[c], (((1,), (1,)), ((), ())),
            preferred_element_type=jnp.float32)


def _fused_body(g0_ref, g1_ref, p0_ref, p1_ref, b_ref, w_ref, o_ref):
    h = jnp.concatenate(
        [g0_ref[0] + p0_ref[0], g1_ref[0] + p1_ref[0]], axis=1)
    h = jnp.maximum(h + b_ref[...], 0.0)
    for c in range(NC):
        o_ref[c] = lax.dot_general(
            h, w_ref[c], (((1,), (1,)), ((), ())),
            preferred_element_type=jnp.float32)


def _final_body(g0_ref, g1_ref, p0_ref, p1_ref, b_ref, o_ref):
    o = jnp.concatenate(
        [g0_ref[0] + p0_ref[0], g1_ref[0] + p1_ref[0]], axis=1)
    o_ref[...] = (o + b_ref[...])[:, :40]


def _matmul_halves(x, wh):
    n, din = x.shape
    dh = wh.shape[1]
    grid = (n // ROW_BLOCK,)
    return pl.pallas_call(
        _mm_body,
        grid=grid,
        in_specs=[
            pl.BlockSpec((ROW_BLOCK, din), lambda i: (i, 0)),
            pl.BlockSpec((NC, dh, din), lambda i: (0, 0, 0)),
        ],
        out_specs=pl.BlockSpec((NC, ROW_BLOCK, dh), lambda i: (0, i, 0)),
        out_shape=jax.ShapeDtypeStruct((NC, n, dh), jnp.float32),
    )(x, wh)


def _fused_update_matmul(g, p, b, wh):
    _, n, dgh = g.shape
    din = 2 * dgh
    dh = wh.shape[1]
    grid = (n // ROW_BLOCK,)
    return pl.pallas_call(
        _fused_body,
        grid=grid,
        in_specs=[
            pl.BlockSpec((1, ROW_BLOCK, dgh), lambda i: (0, i, 0)),
            pl.BlockSpec((1, ROW_BLOCK, dgh), lambda i: (1, i, 0)),
            pl.BlockSpec((1, ROW_BLOCK, dgh), lambda i: (0, i, 0)),
            pl.BlockSpec((1, ROW_BLOCK, dgh), lambda i: (1, i, 0)),
            pl.BlockSpec((1, din), lambda i: (0, 0)),
            pl.BlockSpec((NC, dh, din), lambda i: (0, 0, 0)),
        ],
        out_specs=pl.BlockSpec((NC, ROW_BLOCK, dh), lambda i: (0, i, 0)),
        out_shape=jax.ShapeDtypeStruct((NC, n, dh), jnp.float32),
    )(g, g, p, p, b.reshape(1, din), wh)


def _final_update(g, p, b):
    _, n, dgh = g.shape
    din = 2 * dgh
    grid = (n // ROW_BLOCK,)
    return pl.pallas_call(
        _final_body,
        grid=grid,
        in_specs=[
            pl.BlockSpec((1, ROW_BLOCK, dgh), lambda i: (0, i, 0)),
            pl.BlockSpec((1, ROW_BLOCK, dgh), lambda i: (1, i, 0)),
            pl.BlockSpec((1, ROW_BLOCK, dgh), lambda i: (0, i, 0)),
            pl.BlockSpec((1, ROW_BLOCK, dgh), lambda i: (1, i, 0)),
            pl.BlockSpec((1, din), lambda i: (0, 0)),
        ],
        out_specs=pl.BlockSpec((ROW_BLOCK, 40), lambda i: (i, 0)),
        out_shape=jax.ShapeDtypeStruct((n, 40), jnp.float32),
    )(g, g, p, p, b.reshape(1, din))


@functools.lru_cache(maxsize=None)
def _make_sc_agg(dh):
    mesh = plsc.VectorSubcoreMesh(core_axis_name="c", subcore_axis_name="s")

    @functools.partial(
        pl.kernel,
        out_type=jax.ShapeDtypeStruct((NC, NPAD, dh), jnp.float32),
        mesh=mesh,
        compiler_params=pltpu.CompilerParams(use_tc_tiling_on_sc=False),
        scratch_types=[
            pltpu.VMEM((NB, B), jnp.int32),
            pltpu.VMEM((NB, B), jnp.int32),
            pltpu.VMEM((NBUF, B, dh), jnp.float32),
            pltpu.VMEM_SHARED((NPAD, dh), jnp.float32),
        ]
        + [pltpu.SemaphoreType.DMA] * NBUF
        + [pltpu.SemaphoreType.DMA] * NBUF,
    )
    def agg(table, src_h, dst_h, zrows, out, src_v, dst_v, rows_v, acc,
            *sems):
        sem_g = sems[:NBUF]
        sem_s = sems[NBUF:]
        cid = lax.axis_index("c")
        sid = lax.axis_index("s")
        half = table.at[cid]

        pltpu.sync_copy(zrows, acc.at[pl.ds(sid * RPT, RPT)])
        pltpu.sync_copy(src_h.at[sid], src_v)
        pltpu.sync_copy(dst_h.at[sid], dst_v)
        plsc.subcore_barrier()

        def start_gather(j, b):
            pltpu.async_copy(half.at[src_v.at[j]], rows_v.at[b], sem_g[b])

        def wait_gather(j, b):
            pltpu.make_async_copy(
                half.at[src_v.at[j]], rows_v.at[b], sem_g[b]).wait()

        def start_scatter(j, b):
            pltpu.async_copy(rows_v.at[b], acc.at[dst_v.at[j]], sem_s[b],
                             add=True)

        def wait_scatter(j, b):
            pltpu.make_async_copy(rows_v.at[b], acc.at[dst_v.at[j]],
                                  sem_s[b]).wait()

        for b in range(AHEAD):
            start_gather(b, b)

        def outer(j0, _):
            for b8 in range(NBUF):
                j = j0 * NBUF + b8
                wait_gather(j, b8)
                start_scatter(j, b8)
                jn = j + AHEAD
                bn = (b8 + AHEAD) % NBUF

                @pl.when(jn < NB)
                def _():
                    @pl.when(jn >= NBUF)
                    def _():
                        wait_scatter(jn - NBUF, bn)

                    start_gather(jn, bn)

            return 0

        lax.fori_loop(0, NB // NBUF, outer, 0)

        for j in range(NB - NBUF, NB):
            wait_scatter(j, j % NBUF)

        plsc.subcore_barrier()
        pltpu.sync_copy(acc.at[pl.ds(sid * RPT, RPT)],
                        out.at[cid, pl.ds(sid * RPT, RPT)])

    return agg


def _sc_aggregate(halves, src2, dst2, zrows):
    p = _make_sc_agg(halves.shape[2])(halves, src2, dst2, zrows)
    return p


def kernel(features, adj, W1, b1, W3, b3, W4, b4, W2, b2):
    src2 = adj[0].reshape(NS, NB, B)
    dst2 = adj[1].reshape(NS, NB, B)
    z64 = jnp.zeros((RPT, 64), jnp.float32)
    z32 = jnp.zeros((RPT, 32), jnp.float32)

    W1h = W1.reshape(NC, 64, 128)
    W3h = W3.reshape(NC, 64, 128)
    W4h = W4.reshape(NC, 64, 128)
    W2p = jnp.zeros((64, 128), jnp.float32).at[:40].set(W2).reshape(NC, 32, 128)
    b2p = jnp.zeros((64,), jnp.float32).at[:40].set(b2)

    g1 = _matmul_halves(features, W1h)
    p1 = _sc_aggregate(g1, src2, dst2, z64)
    g2 = _fused_update_matmul(g1, p1, b1, W3h)
    p2 = _sc_aggregate(g2, src2, dst2, z64)
    g3 = _fused_update_matmul(g2, p2, b3, W4h)
    p3 = _sc_aggregate(g3, src2, dst2, z64)
    g4 = _fused_update_matmul(g3, p3, b4, W2p)
    p4 = _sc_aggregate(g4, src2, dst2, z32)
    return _final_update(g4, p4, b2p)


# speedup vs baseline: 1.9117x; 1.0360x over previous
import functools

import jax
import jax.numpy as jnp
from jax import lax
from jax.experimental import pallas as pl
from jax.experimental.pallas import tpu as pltpu
from jax.experimental.pallas import tpu_sc as plsc

N_NODES = 10000
N_EDGES = 320000

NC = 2
NS = 16
EW = N_EDGES // NS
B = 125
NB = EW // B
NBUF = 5
AHEAD = 4
NPAD = 10240
RPT = NPAD // NS

ROW_BLOCK = 2000


def _mm_body(x_ref, w_ref, o_ref):
    x = x_ref[...]
    for c in range(NC):
        o_ref[c] = lax.dot_general(
            x, w_ref[c], (((1,), (1,)), ((), ())),
            preferred_element_type=jnp.float32)


def _fused_body(g0_ref, g1_ref, p0_ref, p1_ref, b_ref, w_ref, o_ref):
    h = jnp.concatenate(
        [g0_ref[0] + p0_ref[0], g1_ref[0] + p1_ref[0]], axis=1)
    h = jnp.maximum(h + b_ref[...], 0.0)
    for c in range(NC):
        o_ref[c] = lax.dot_general(
            h, w_ref[c], (((1,), (1,)), ((), ())),
            preferred_element_type=jnp.float32)


def _final_body(g0_ref, g1_ref, p0_ref, p1_ref, b_ref, o_ref):
    o = jnp.concatenate(
        [g0_ref[0] + p0_ref[0], g1_ref[0] + p1_ref[0]], axis=1)
    o_ref[...] = (o + b_ref[...])[:, :40]


def _matmul_halves(x, wh):
    n, din = x.shape
    dh = wh.shape[1]
    grid = (n // ROW_BLOCK,)
    return pl.pallas_call(
        _mm_body,
        grid=grid,
        in_specs=[
            pl.BlockSpec((ROW_BLOCK, din), lambda i: (i, 0)),
            pl.BlockSpec((NC, dh, din), lambda i: (0, 0, 0)),
        ],
        out_specs=pl.BlockSpec((NC, ROW_BLOCK, dh), lambda i: (0, i, 0)),
        out_shape=jax.ShapeDtypeStruct((NC, n, dh), jnp.float32),
    )(x, wh)


def _fused_update_matmul(g, p, b, wh):
    _, n, dgh = g.shape
    din = 2 * dgh
    dh = wh.shape[1]
    grid = (n // ROW_BLOCK,)
    return pl.pallas_call(
        _fused_body,
        grid=grid,
        in_specs=[
            pl.BlockSpec((1, ROW_BLOCK, dgh), lambda i: (0, i, 0)),
            pl.BlockSpec((1, ROW_BLOCK, dgh), lambda i: (1, i, 0)),
            pl.BlockSpec((1, ROW_BLOCK, dgh), lambda i: (0, i, 0)),
            pl.BlockSpec((1, ROW_BLOCK, dgh), lambda i: (1, i, 0)),
            pl.BlockSpec((1, din), lambda i: (0, 0)),
            pl.BlockSpec((NC, dh, din), lambda i: (0, 0, 0)),
        ],
        out_specs=pl.BlockSpec((NC, ROW_BLOCK, dh), lambda i: (0, i, 0)),
        out_shape=jax.ShapeDtypeStruct((NC, n, dh), jnp.float32),
    )(g, g, p, p, b.reshape(1, din), wh)


def _final_update(g, p, b):
    _, n, dgh = g.shape
    din = 2 * dgh
    grid = (n // ROW_BLOCK,)
    return pl.pallas_call(
        _final_body,
        grid=grid,
        in_specs=[
            pl.BlockSpec((1, ROW_BLOCK, dgh), lambda i: (0, i, 0)),
            pl.BlockSpec((1, ROW_BLOCK, dgh), lambda i: (1, i, 0)),
            pl.BlockSpec((1, ROW_BLOCK, dgh), lambda i: (0, i, 0)),
            pl.BlockSpec((1, ROW_BLOCK, dgh), lambda i: (1, i, 0)),
            pl.BlockSpec((1, din), lambda i: (0, 0)),
        ],
        out_specs=pl.BlockSpec((ROW_BLOCK, 40), lambda i: (i, 0)),
        out_shape=jax.ShapeDtypeStruct((n, 40), jnp.float32),
    )(g, g, p, p, b.reshape(1, din))


@functools.lru_cache(maxsize=None)
def _make_sc_agg(dh):
    mesh = plsc.VectorSubcoreMesh(core_axis_name="c", subcore_axis_name="s")

    @functools.partial(
        pl.kernel,
        out_type=jax.ShapeDtypeStruct((NC, NPAD, dh), jnp.float32),
        mesh=mesh,
        compiler_params=pltpu.CompilerParams(use_tc_tiling_on_sc=False),
        scratch_types=[
            pltpu.VMEM((NB, B), jnp.int32),
            pltpu.VMEM((NB, B), jnp.int32),
            pltpu.VMEM((NBUF, B, dh), jnp.float32),
            pltpu.VMEM_SHARED((NPAD, dh), jnp.float32),
        ]
        + [pltpu.SemaphoreType.DMA] * NBUF
        + [pltpu.SemaphoreType.DMA] * NBUF,
    )
    def agg(table, src_h, dst_h, zrows, out, src_v, dst_v, rows_v, acc,
            *sems):
        sem_g = sems[:NBUF]
        sem_s = sems[NBUF:]
        cid = lax.axis_index("c")
        sid = lax.axis_index("s")
        half = table.at[cid]

        pltpu.sync_copy(zrows, acc.at[pl.ds(sid * RPT, RPT)])
        pltpu.sync_copy(src_h.at[sid], src_v)
        pltpu.sync_copy(dst_h.at[sid], dst_v)
        plsc.subcore_barrier()

        def start_gather(j, b):
            pltpu.async_copy(half.at[src_v.at[j]], rows_v.at[b], sem_g[b])

        def wait_gather(j, b):
            pltpu.make_async_copy(
                half.at[src_v.at[j]], rows_v.at[b], sem_g[b]).wait()

        def start_scatter(j, b):
            pltpu.async_copy(rows_v.at[b], acc.at[dst_v.at[j]], sem_s[b],
                             add=True)

        def wait_scatter(j, b):
            pltpu.make_async_copy(rows_v.at[b], acc.at[dst_v.at[j]],
                                  sem_s[b]).wait()

        for b in range(AHEAD):
            start_gather(b, b)

        def outer(j0, _):
            for b8 in range(NBUF):
                j = j0 * NBUF + b8
                wait_gather(j, b8)
                start_scatter(j, b8)
                jn = j + AHEAD
                bn = (b8 + AHEAD) % NBUF

                @pl.when(jn < NB)
                def _():
                    @pl.when(jn >= NBUF)
                    def _():
                        wait_scatter(jn - NBUF, bn)

                    start_gather(jn, bn)

            return 0

        lax.fori_loop(0, NB // NBUF, outer, 0)

        for j in range(NB - NBUF, NB):
            wait_scatter(j, j % NBUF)

        plsc.subcore_barrier()
        pltpu.sync_copy(acc.at[pl.ds(sid * RPT, RPT)],
                        out.at[cid, pl.ds(sid * RPT, RPT)])

    return agg


def _sc_aggregate(halves, src2, dst2, zrows):
    p = _make_sc_agg(halves.shape[2])(halves, src2, dst2, zrows)
    return p


def kernel(features, adj, W1, b1, W3, b3, W4, b4, W2, b2):
    src2 = adj[0].reshape(NS, NB, B)
    dst2 = adj[1].reshape(NS, NB, B)
    z64 = jnp.zeros((RPT, 64), jnp.float32)
    z32 = jnp.zeros((RPT, 32), jnp.float32)

    W1h = W1.reshape(NC, 64, 128)
    W3h = W3.reshape(NC, 64, 128)
    W4h = W4.reshape(NC, 64, 128)
    W2p = jnp.zeros((64, 128), jnp.float32).at[:40].set(W2).reshape(NC, 32, 128)
    b2p = jnp.zeros((64,), jnp.float32).at[:40].set(b2)

    g1 = _matmul_halves(features, W1h)
    p1 = _sc_aggregate(g1, src2, dst2, z64)
    g2 = _fused_update_matmul(g1, p1, b1, W3h)
    p2 = _sc_aggregate(g2, src2, dst2, z64)
    g3 = _fused_update_matmul(g2, p2, b3, W4h)
    p3 = _sc_aggregate(g3, src2, dst2, z64)
    g4 = _fused_update_matmul(g3, p3, b4, W2p)
    p4 = _sc_aggregate(g4, src2, dst2, z32)
    return _final_update(g4, p4, b2p)
